# baseline (speedup 1.0000x reference)
import functools

import jax
import jax.numpy as jnp
from jax import lax
from jax.experimental import pallas as pl
from jax.experimental.pallas import tpu as pltpu
from jax.experimental.pallas import tpu_sc as plsc

E = 8
K = 2
D = 1024
C = 1024
B = 2
N = 2048
T = B * N
A = K * T
BLK = 512
CAP = A + E * BLK
NBLK = CAP // BLK
TT = 512
NT = T // TT
TILES_PER_B = NT // B
AB = 512
NAB = A // AB

NC = 2
NS = 16
NW = NC * NS

_F32 = jnp.float32
_I32 = jnp.int32
_BF16 = jnp.bfloat16


def _erf(x):
    s = jnp.sign(x)
    a = jnp.abs(x)
    t = 1.0 / (1.0 + 0.3275911 * a)
    poly = t * (0.254829592
                + t * (-0.284496736
                       + t * (1.421413741
                              + t * (-1.453152027 + t * 1.061405429))))
    return s * (1.0 - poly * jnp.exp(-a * a))


def _gelu_exact(x):
    return 0.5 * x * (1.0 + _erf(x * 0.7071067811865476))


def _gate_kernel(x_ref, gw_ref, e1_ref, e2_ref, w1n_ref, w2n_ref, bal_ref,
                 z_ref, slots_ref, be_ref, act_ref, proxy_acc, dens_acc,
                 z_acc, cnt_acc, esel_acc):
    i = pl.program_id(0)
    x = x_ref[...]
    gw = gw_ref[...]
    logits = jax.lax.dot_general(x, gw, (((1,), (1,)), ((), ())),
                                 preferred_element_type=_F32)
    m = jnp.max(logits, axis=1, keepdims=True)
    ex = jnp.exp(logits - m)
    sumex = jnp.sum(ex, axis=1, keepdims=True)
    lse = m[:, 0] + jnp.log(sumex[:, 0])
    p = ex / sumex

    lane = jax.lax.broadcasted_iota(_I32, p.shape, 1)
    m1 = jnp.max(p, axis=1, keepdims=True)
    i1 = jnp.min(jnp.where(p == m1, lane, E), axis=1, keepdims=True)
    oh1 = lane == i1
    p2 = jnp.where(oh1, -jnp.inf, p)
    m2 = jnp.max(p2, axis=1, keepdims=True)
    i2 = jnp.min(jnp.where(p2 == m2, lane, E), axis=1, keepdims=True)
    oh2 = lane == i2
    denom = m1 + m2
    e1_ref[...] = i1
    e2_ref[...] = i2
    w1n_ref[...] = m1 / denom
    w2n_ref[...] = m2 / denom
    esel_acc[pl.ds(i * TT, TT), :] = i1
    esel_acc[pl.ds(T + i * TT, TT), :] = i2

    @pl.when(i == 0)
    def _():
        proxy_acc[...] = jnp.zeros_like(proxy_acc)
        dens_acc[...] = jnp.zeros_like(dens_acc)
        z_acc[...] = jnp.zeros_like(z_acc)
        cnt_acc[...] = jnp.zeros_like(cnt_acc)

    b = i // TILES_PER_B
    rows = jax.lax.broadcasted_iota(_I32, (B, E), 0)
    sel = (rows == b).astype(_F32)
    proxy_acc[...] += sel * jnp.sum(p, axis=0)[None, :]
    dens_acc[...] += sel * jnp.sum(oh1.astype(_F32), axis=0)[None, :]
    z_acc[...] += jnp.sum(lse * lse).reshape(1, 1)
    cnt_acc[...] += (jnp.sum(oh1.astype(_F32), axis=0)
                     + jnp.sum(oh2.astype(_F32), axis=0))[None, :]

    @pl.when(i == NT - 1)
    def _():
        proxy = proxy_acc[...] / jnp.float32(N)
        dens = dens_acc[...] / jnp.float32(N)
        bal_ref[...] = (jnp.sum(proxy * dens) / jnp.float32(B * E)
                        * jnp.float32(E * E)).reshape(1, 1)
        z_ref[...] = (z_acc[0, 0] / jnp.float32(T)).reshape(1, 1)

        off = []
        run = jnp.float32(0.0)
        maxe = jnp.float32(0.0)
        fBLK = jnp.float32(BLK)
        for e in range(E):
            off.append(run)
            pad_e = jnp.floor((cnt_acc[0, e] + (BLK - 1)) / fBLK) * fBLK
            maxe = jnp.where(pad_e > 0, jnp.float32(e), maxe)
            run = run + pad_e
        starts = (jax.lax.broadcasted_iota(_I32, (1, NBLK), 1)
                  .astype(_F32) * fBLK)
        be = jnp.zeros((1, NBLK), _F32)
        for e in range(E):
            pad_e = jnp.floor((cnt_acc[0, e] + (BLK - 1)) / fBLK) * fBLK
            be = be + (starts >= off[e] + pad_e).astype(_F32)
        be_ref[...] = jnp.minimum(be, maxe).astype(_I32)
        act_ref[...] = (starts < run).astype(_I32)

        r = jax.lax.broadcasted_iota(_I32, (AB, AB), 0)
        c = jax.lax.broadcasted_iota(_I32, (AB, AB), 1)
        tril = (c < r).astype(_F32)
        runf = [jnp.float32(0.0)] * E
        for g in range(NAB):
            ev = esel_acc[pl.ds(g * AB, AB), :]
            onehot = (ev == jax.lax.broadcasted_iota(
                _I32, (AB, E), 1)).astype(_F32)
            rank = jax.lax.dot_general(tril, onehot,
                                       (((1,), (0,)), ((), ())),
                                       preferred_element_type=_F32)
            rank_row = jnp.sum(rank * onehot, axis=1, keepdims=True)
            base = jnp.zeros((AB, 1), _F32)
            for e in range(E):
                base = base + jnp.where(ev == e, off[e] + runf[e], 0.0)
                runf[e] = runf[e] + jnp.sum(onehot[:, e])
            slots_ref[pl.ds(g * AB, AB), :] = (base + rank_row).astype(_I32)


def _gate(x2d, gate_w):
    return pl.pallas_call(
        _gate_kernel,
        grid=(NT,),
        in_specs=[
            pl.BlockSpec((TT, D), lambda i: (i, 0)),
            pl.BlockSpec((E, D), lambda i: (0, 0)),
        ],
        out_specs=[
            pl.BlockSpec((TT, 1), lambda i: (i, 0)),
            pl.BlockSpec((TT, 1), lambda i: (i, 0)),
            pl.BlockSpec((TT, 1), lambda i: (i, 0)),
            pl.BlockSpec((TT, 1), lambda i: (i, 0)),
            pl.BlockSpec((1, 1), lambda i: (0, 0)),
            pl.BlockSpec((1, 1), lambda i: (0, 0)),
            pl.BlockSpec((A, 1), lambda i: (0, 0)),
            pl.BlockSpec((1, NBLK), lambda i: (0, 0)),
            pl.BlockSpec((1, NBLK), lambda i: (0, 0)),
        ],
        out_shape=[
            jax.ShapeDtypeStruct((T, 1), _I32),
            jax.ShapeDtypeStruct((T, 1), _I32),
            jax.ShapeDtypeStruct((T, 1), _F32),
            jax.ShapeDtypeStruct((T, 1), _F32),
            jax.ShapeDtypeStruct((1, 1), _F32),
            jax.ShapeDtypeStruct((1, 1), _F32),
            jax.ShapeDtypeStruct((A, 1), _I32),
            jax.ShapeDtypeStruct((1, NBLK), _I32),
            jax.ShapeDtypeStruct((1, NBLK), _I32),
        ],
        scratch_shapes=[
            pltpu.VMEM((B, E), _F32),
            pltpu.VMEM((B, E), _F32),
            pltpu.VMEM((1, 1), _F32),
            pltpu.VMEM((1, E), _F32),
            pltpu.VMEM((A, 1), _I32),
        ],
    )(x2d, gate_w)


_A_PER_W = A // NW
_DCH = 32
_NDCH = _A_PER_W // _DCH


@functools.cache
def _vmesh():
    return plsc.VectorSubcoreMesh(core_axis_name="c", subcore_axis_name="s")


@functools.cache
def _dispatch_kernel():
    @functools.partial(
        pl.kernel,
        mesh=_vmesh(),
        out_type=jax.ShapeDtypeStruct((CAP, D), _F32),
        scratch_types=[
            pltpu.VMEM((_DCH,), _I32),
            pltpu.VMEM((_DCH,), _I32),
            pltpu.VMEM((_DCH,), _I32),
            pltpu.VMEM((_DCH,), _I32),
            pltpu.VMEM((_DCH, D), _F32),
            pltpu.VMEM((_DCH, D), _F32),
            pltpu.SemaphoreType.DMA,
            pltpu.SemaphoreType.DMA,
            pltpu.SemaphoreType.DMA,
            pltpu.SemaphoreType.DMA,
        ],
    )
    def _dispatch(slots_hbm, tok_hbm, x_hbm, xg_hbm, t0, t1, s0, s1,
                  rows0, rows1, g0, g1, w0, w1):
        sid = lax.axis_index("s")
        cid = lax.axis_index("c")
        wid = sid * NC + cid
        abase = wid * _A_PER_W
        tokb = (t0, t1)
        slotb = (s0, s1)
        rowb = (rows0, rows1)
        gsem = (g0, g1)
        wsem = (w0, w1)
        ghandles = [None, None]
        whandles = [None, None]
        for j in range(_NDCH):
            b = j % 2
            off = abase + j * _DCH
            if whandles[b] is not None:
                whandles[b].wait()
            pltpu.sync_copy(tok_hbm.at[pl.ds(off, _DCH)], tokb[b])
            pltpu.sync_copy(slots_hbm.at[pl.ds(off, _DCH)], slotb[b])
            ghandles[b] = pltpu.async_copy(x_hbm.at[tokb[b]], rowb[b],
                                           gsem[b])
            if j >= 1:
                pb = 1 - b
                ghandles[pb].wait()
                whandles[pb] = pltpu.async_copy(rowb[pb],
                                                xg_hbm.at[slotb[pb]],
                                                wsem[pb])
        lb = (_NDCH - 1) % 2
        ghandles[lb].wait()
        whandles[lb] = pltpu.async_copy(rowb[lb], xg_hbm.at[slotb[lb]],
                                        wsem[lb])
        whandles[0].wait()
        whandles[1].wait()

    return _dispatch


def _mlp_kernel(be_ref, act_ref, xg_ref, w1_ref, b1_ref, w2_ref, b2_ref,
                yg_ref):
    @pl.when(act_ref[pl.program_id(0)] == 1)
    def _():
        h = jax.lax.dot_general(xg_ref[...], w1_ref[0],
                                (((1,), (1,)), ((), ())),
                                preferred_element_type=_F32)
        h = h + b1_ref[0]
        h = _gelu_exact(h)
        o = jax.lax.dot_general(h, w2_ref[0], (((1,), (1,)), ((), ())),
                                preferred_element_type=_F32)
        yg_ref[...] = o + b2_ref[0]


def _mlp(block_expert, act, xg, w1, b1, w2, b2):
    grid_spec = pltpu.PrefetchScalarGridSpec(
        num_scalar_prefetch=2,
        grid=(NBLK,),
        in_specs=[
            pl.BlockSpec((BLK, D),
                         lambda i, be, act: (act[i] * i, 0)),
            pl.BlockSpec((1, C, D), lambda i, be, act: (be[i], 0, 0)),
            pl.BlockSpec((1, 1, C), lambda i, be, act: (be[i], 0, 0)),
            pl.BlockSpec((1, C, C), lambda i, be, act: (be[i], 0, 0)),
            pl.BlockSpec((1, 1, C), lambda i, be, act: (be[i], 0, 0)),
        ],
        out_specs=pl.BlockSpec(
            (BLK, C), lambda i, be, act: (jnp.where(act[i] == 1, i, NBLK), 0)),
    )
    return pl.pallas_call(
        _mlp_kernel,
        grid_spec=grid_spec,
        out_shape=jax.ShapeDtypeStruct((CAP + BLK, C), _F32),
    )(block_expert, act, xg, w1, b1.reshape(E, 1, C), w2, b2.reshape(E, 1, C))


_T_PER_W = T // NW
_CCH = 32
_NCCH = _T_PER_W // _CCH


@functools.cache
def _combine_kernel():
    @functools.partial(
        pl.kernel,
        mesh=_vmesh(),
        out_type=[
            jax.ShapeDtypeStruct((T, C), _F32),
            jax.ShapeDtypeStruct((T, C), _F32),
        ],
        scratch_types=[
            pltpu.VMEM((_CCH,), _I32),
            pltpu.VMEM((_CCH,), _I32),
            pltpu.VMEM((_CCH, C), _F32),
            pltpu.VMEM((_CCH, C), _F32),
            pltpu.SemaphoreType.DMA,
            pltpu.SemaphoreType.DMA,
            pltpu.SemaphoreType.DMA,
            pltpu.SemaphoreType.DMA,
        ],
    )
    def _combine(yg_hbm, s1_hbm, s2_hbm, y1_hbm, y2_hbm, i0, i1, v0, v1,
                 g0, g1, w0, w1):
        sid = lax.axis_index("s")
        cid = lax.axis_index("c")
        wid = sid * NC + cid
        tbase = wid * _T_PER_W
        tasks = []
        for j in range(_NCCH):
            off = tbase + j * _CCH
            tasks.append((s1_hbm, y1_hbm, off))
            tasks.append((s2_hbm, y2_hbm, off))
        idxb = (i0, i1)
        rowb = (v0, v1)
        gsem = (g0, g1)
        wsem = (w0, w1)
        ghandles = [None, None]
        whandles = [None, None]
        nt = len(tasks)
        for t, (sh, yh, off) in enumerate(tasks):
            b = t % 2
            if whandles[b] is not None:
                whandles[b].wait()
            pltpu.sync_copy(sh.at[pl.ds(off, _CCH)], idxb[b])
            ghandles[b] = pltpu.async_copy(yg_hbm.at[idxb[b]], rowb[b],
                                           gsem[b])
            if t >= 1:
                pb = 1 - b
                psh, pyh, poff = tasks[t - 1]
                ghandles[pb].wait()
                whandles[pb] = pltpu.async_copy(
                    rowb[pb], pyh.at[pl.ds(poff, _CCH)], wsem[pb])
        lb = (nt - 1) % 2
        ghandles[lb].wait()
        lsh, lyh, loff = tasks[nt - 1]
        whandles[lb] = pltpu.async_copy(rowb[lb], lyh.at[pl.ds(loff, _CCH)],
                                        wsem[lb])
        whandles[0].wait()
        whandles[1].wait()

    return _combine


def _finish_kernel(y1_ref, y2_ref, w1n_ref, w2n_ref, out_ref):
    out_ref[...] = (w1n_ref[...] * y1_ref[...]
                    + w2n_ref[...] * y2_ref[...])


def _finish(y1, y2, w1n, w2n):
    return pl.pallas_call(
        _finish_kernel,
        grid=(NT,),
        in_specs=[
            pl.BlockSpec((TT, C), lambda i: (i, 0)),
            pl.BlockSpec((TT, C), lambda i: (i, 0)),
            pl.BlockSpec((TT, 1), lambda i: (i, 0)),
            pl.BlockSpec((TT, 1), lambda i: (i, 0)),
        ],
        out_specs=pl.BlockSpec((TT, C), lambda i: (i, 0)),
        out_shape=jax.ShapeDtypeStruct((T, C), _F32),
    )(y1, y2, w1n, w2n)


@jax.jit
def kernel(x_img, gate_w, w1, b1, w2, b2):
    x2d = x_img.reshape(T, D)
    e1, e2, w1n, w2n, bal, z, slots, be, act = _gate(x2d, gate_w)
    slots_flat = slots.reshape(A)
    tokvals = jnp.tile(jnp.arange(T, dtype=_I32), 2)
    xg = _dispatch_kernel()(slots_flat, tokvals, x2d)
    yg = _mlp(be.reshape(NBLK), act.reshape(NBLK), xg, w1, b1, w2, b2)
    y1, y2 = _combine_kernel()(yg, slots_flat[:T], slots_flat[T:])
    out = _finish(y1, y2, w1n, w2n)
    return (out.reshape(B, N, C), bal[0, 0], z[0, 0])


# speedup vs baseline: 1.0766x; 1.0766x over previous
import functools

import jax
import jax.numpy as jnp
from jax import lax
from jax.experimental import pallas as pl
from jax.experimental.pallas import tpu as pltpu
from jax.experimental.pallas import tpu_sc as plsc

E = 8
K = 2
D = 1024
C = 1024
B = 2
N = 2048
T = B * N
A = K * T
BLK = 512
CAP = A + E * BLK
NBLK = CAP // BLK
TT = 512
NT = T // TT
TILES_PER_B = NT // B
AB = 512
NAB = A // AB

NC = 2
NS = 16
NW = NC * NS

_F32 = jnp.float32
_I32 = jnp.int32
_BF16 = jnp.bfloat16


def _erf(x):
    s = jnp.sign(x)
    a = jnp.abs(x)
    t = 1.0 / (1.0 + 0.3275911 * a)
    poly = t * (0.254829592
                + t * (-0.284496736
                       + t * (1.421413741
                              + t * (-1.453152027 + t * 1.061405429))))
    return s * (1.0 - poly * jnp.exp(-a * a))


def _gelu_exact(x):
    return 0.5 * x * (1.0 + _erf(x * 0.7071067811865476))


def _gate_kernel(x_ref, gw_ref, e1_ref, e2_ref, w1n_ref, w2n_ref, bal_ref,
                 z_ref, cnt_ref, proxy_acc, dens_acc, z_acc, cnt_acc):
    i = pl.program_id(0)
    x = x_ref[...]
    gw = gw_ref[...]
    logits = jax.lax.dot_general(x, gw, (((1,), (1,)), ((), ())),
                                 preferred_element_type=_F32)
    m = jnp.max(logits, axis=1, keepdims=True)
    ex = jnp.exp(logits - m)
    sumex = jnp.sum(ex, axis=1, keepdims=True)
    lse = m[:, 0] + jnp.log(sumex[:, 0])
    p = ex / sumex

    lane = jax.lax.broadcasted_iota(_I32, p.shape, 1)
    m1 = jnp.max(p, axis=1, keepdims=True)
    i1 = jnp.min(jnp.where(p == m1, lane, E), axis=1, keepdims=True)
    oh1 = lane == i1
    p2 = jnp.where(oh1, -jnp.inf, p)
    m2 = jnp.max(p2, axis=1, keepdims=True)
    i2 = jnp.min(jnp.where(p2 == m2, lane, E), axis=1, keepdims=True)
    oh2 = lane == i2
    denom = m1 + m2
    e1_ref[...] = i1
    e2_ref[...] = i2
    w1n_ref[...] = m1 / denom
    w2n_ref[...] = m2 / denom

    @pl.when(i == 0)
    def _():
        proxy_acc[...] = jnp.zeros_like(proxy_acc)
        dens_acc[...] = jnp.zeros_like(dens_acc)
        z_acc[...] = jnp.zeros_like(z_acc)
        cnt_acc[...] = jnp.zeros_like(cnt_acc)

    b = i // TILES_PER_B
    rows = jax.lax.broadcasted_iota(_I32, (B, E), 0)
    sel = (rows == b).astype(_F32)
    proxy_acc[...] += sel * jnp.sum(p, axis=0)[None, :]
    dens_acc[...] += sel * jnp.sum(oh1.astype(_F32), axis=0)[None, :]
    z_acc[...] += jnp.sum(lse * lse).reshape(1, 1)
    cnt_acc[...] += (jnp.sum(oh1.astype(_F32), axis=0)
                     + jnp.sum(oh2.astype(_F32), axis=0))[None, :]

    @pl.when(i == NT - 1)
    def _():
        proxy = proxy_acc[...] / jnp.float32(N)
        dens = dens_acc[...] / jnp.float32(N)
        bal_ref[...] = (jnp.sum(proxy * dens) / jnp.float32(B * E)
                        * jnp.float32(E * E)).reshape(1, 1)
        z_ref[...] = (z_acc[0, 0] / jnp.float32(T)).reshape(1, 1)
        cnt_ref[...] = cnt_acc[...].astype(_I32)


def _gate(x2d, gate_w):
    return pl.pallas_call(
        _gate_kernel,
        grid=(NT,),
        in_specs=[
            pl.BlockSpec((TT, D), lambda i: (i, 0)),
            pl.BlockSpec((E, D), lambda i: (0, 0)),
        ],
        out_specs=[
            pl.BlockSpec((TT, 1), lambda i: (i, 0)),
            pl.BlockSpec((TT, 1), lambda i: (i, 0)),
            pl.BlockSpec((TT, 1), lambda i: (i, 0)),
            pl.BlockSpec((TT, 1), lambda i: (i, 0)),
            pl.BlockSpec((1, 1), lambda i: (0, 0)),
            pl.BlockSpec((1, 1), lambda i: (0, 0)),
            pl.BlockSpec((1, E), lambda i: (0, 0)),
        ],
        out_shape=[
            jax.ShapeDtypeStruct((T, 1), _I32),
            jax.ShapeDtypeStruct((T, 1), _I32),
            jax.ShapeDtypeStruct((T, 1), _F32),
            jax.ShapeDtypeStruct((T, 1), _F32),
            jax.ShapeDtypeStruct((1, 1), _F32),
            jax.ShapeDtypeStruct((1, 1), _F32),
            jax.ShapeDtypeStruct((1, E), _I32),
        ],
        scratch_shapes=[
            pltpu.VMEM((B, E), _F32),
            pltpu.VMEM((B, E), _F32),
            pltpu.VMEM((1, 1), _F32),
            pltpu.VMEM((1, E), _F32),
        ],
    )(x2d, gate_w)


_MG = 16
_MGW = A // _MG


def _meta_kernel(cnt_ref, esel_ref, slots_ref, be_ref, act_ref):
    off = []
    run = 0
    for e in range(E):
        off.append(run)
        pad_e = ((cnt_ref[e] + (BLK - 1)) // BLK) * BLK
        run = run + pad_e
    starts = jax.lax.broadcasted_iota(_I32, (1, NBLK), 1) * BLK
    be = jnp.zeros((1, NBLK), _I32)
    maxe = 0
    for e in range(E):
        pad_e = ((cnt_ref[e] + (BLK - 1)) // BLK) * BLK
        end_e = off[e] + pad_e
        be = be + (starts >= end_e).astype(_I32)
        maxe = jnp.where(pad_e > 0, e, maxe)
    be_ref[...] = jnp.minimum(be, maxe)
    act_ref[...] = (starts < run).astype(_I32)

    r = jax.lax.broadcasted_iota(_I32, (_MGW, _MGW), 0)
    c = jax.lax.broadcasted_iota(_I32, (_MGW, _MGW), 1)
    triu = (r < c).astype(_F32)
    sub = jax.lax.broadcasted_iota(_I32, (E, _MGW), 0)
    runf = [jnp.float32(0.0)] * E
    for g in range(_MG):
        ev = esel_ref[g, :][None, :]
        onehot = (jnp.broadcast_to(ev, (E, _MGW)) == sub).astype(_F32)
        rank = jax.lax.dot_general(onehot, triu, (((1,), (0,)), ((), ())),
                                   preferred_element_type=_F32)
        rank_row = jnp.sum(rank * onehot, axis=0, keepdims=True)
        base = jnp.zeros((1, _MGW), _F32)
        for e in range(E):
            base = base + jnp.where(ev == e,
                                    jnp.float32(off[e]) + runf[e], 0.0)
            runf[e] = runf[e] + jnp.sum(onehot[e, :])
        slots_ref[g, :] = (base + rank_row).astype(_I32)[0, :]


def _meta(cnt, esel2d):
    grid_spec = pltpu.PrefetchScalarGridSpec(
        num_scalar_prefetch=1,
        grid=(1,),
        in_specs=[
            pl.BlockSpec((_MG, _MGW), lambda i, cnt: (0, 0)),
        ],
        out_specs=[
            pl.BlockSpec((_MG, _MGW), lambda i, cnt: (0, 0)),
            pl.BlockSpec((1, NBLK), lambda i, cnt: (0, 0)),
            pl.BlockSpec((1, NBLK), lambda i, cnt: (0, 0)),
        ],
    )
    return pl.pallas_call(
        _meta_kernel,
        grid_spec=grid_spec,
        out_shape=[
            jax.ShapeDtypeStruct((_MG, _MGW), _I32),
            jax.ShapeDtypeStruct((1, NBLK), _I32),
            jax.ShapeDtypeStruct((1, NBLK), _I32),
        ],
    )(cnt, esel2d)


_A_PER_W = A // NW
_DCH = 32
_NDCH = _A_PER_W // _DCH


@functools.cache
def _vmesh():
    return plsc.VectorSubcoreMesh(core_axis_name="c", subcore_axis_name="s")


@functools.cache
def _dispatch_kernel():
    @functools.partial(
        pl.kernel,
        mesh=_vmesh(),
        out_type=jax.ShapeDtypeStruct((CAP, D), _F32),
        scratch_types=[
            pltpu.VMEM((_DCH,), _I32),
            pltpu.VMEM((_DCH,), _I32),
            pltpu.VMEM((_DCH,), _I32),
            pltpu.VMEM((_DCH,), _I32),
            pltpu.VMEM((_DCH, D), _F32),
            pltpu.VMEM((_DCH, D), _F32),
            pltpu.SemaphoreType.DMA,
            pltpu.SemaphoreType.DMA,
            pltpu.SemaphoreType.DMA,
            pltpu.SemaphoreType.DMA,
        ],
    )
    def _dispatch(slots_hbm, tok_hbm, x_hbm, xg_hbm, t0, t1, s0, s1,
                  rows0, rows1, g0, g1, w0, w1):
        sid = lax.axis_index("s")
        cid = lax.axis_index("c")
        wid = sid * NC + cid
        abase = wid * _A_PER_W
        tokb = (t0, t1)
        slotb = (s0, s1)
        rowb = (rows0, rows1)
        gsem = (g0, g1)
        wsem = (w0, w1)
        ghandles = [None, None]
        whandles = [None, None]
        for j in range(_NDCH):
            b = j % 2
            off = abase + j * _DCH
            if whandles[b] is not None:
                whandles[b].wait()
            pltpu.sync_copy(tok_hbm.at[pl.ds(off, _DCH)], tokb[b])
            pltpu.sync_copy(slots_hbm.at[pl.ds(off, _DCH)], slotb[b])
            ghandles[b] = pltpu.async_copy(x_hbm.at[tokb[b]], rowb[b],
                                           gsem[b])
            if j >= 1:
                pb = 1 - b
                ghandles[pb].wait()
                whandles[pb] = pltpu.async_copy(rowb[pb],
                                                xg_hbm.at[slotb[pb]],
                                                wsem[pb])
        lb = (_NDCH - 1) % 2
        ghandles[lb].wait()
        whandles[lb] = pltpu.async_copy(rowb[lb], xg_hbm.at[slotb[lb]],
                                        wsem[lb])
        whandles[0].wait()
        whandles[1].wait()

    return _dispatch


def _mlp_kernel(be_ref, act_ref, xg_ref, w1_ref, b1_ref, w2_ref, b2_ref,
                yg_ref):
    @pl.when(act_ref[pl.program_id(0)] == 1)
    def _():
        h = jax.lax.dot_general(xg_ref[...], w1_ref[0],
                                (((1,), (1,)), ((), ())),
                                preferred_element_type=_F32)
        h = h + b1_ref[0]
        h = _gelu_exact(h)
        o = jax.lax.dot_general(h, w2_ref[0], (((1,), (1,)), ((), ())),
                                preferred_element_type=_F32)
        yg_ref[...] = o + b2_ref[0]


def _mlp(block_expert, act, xg, w1, b1, w2, b2):
    grid_spec = pltpu.PrefetchScalarGridSpec(
        num_scalar_prefetch=2,
        grid=(NBLK,),
        in_specs=[
            pl.BlockSpec((BLK, D),
                         lambda i, be, act: (act[i] * i, 0)),
            pl.BlockSpec((1, C, D), lambda i, be, act: (be[i], 0, 0)),
            pl.BlockSpec((1, 1, C), lambda i, be, act: (be[i], 0, 0)),
            pl.BlockSpec((1, C, C), lambda i, be, act: (be[i], 0, 0)),
            pl.BlockSpec((1, 1, C), lambda i, be, act: (be[i], 0, 0)),
        ],
        out_specs=pl.BlockSpec(
            (BLK, C), lambda i, be, act: (jnp.where(act[i] == 1, i, NBLK), 0)),
    )
    return pl.pallas_call(
        _mlp_kernel,
        grid_spec=grid_spec,
        out_shape=jax.ShapeDtypeStruct((CAP + BLK, C), _F32),
    )(block_expert, act, xg, w1, b1.reshape(E, 1, C), w2, b2.reshape(E, 1, C))


_T_PER_W = T // NW
_CCH = 32
_NCCH = _T_PER_W // _CCH


@functools.cache
def _combine_kernel():
    @functools.partial(
        pl.kernel,
        mesh=_vmesh(),
        out_type=[
            jax.ShapeDtypeStruct((T, C), _F32),
            jax.ShapeDtypeStruct((T, C), _F32),
        ],
        scratch_types=[
            pltpu.VMEM((_CCH,), _I32),
            pltpu.VMEM((_CCH,), _I32),
            pltpu.VMEM((_CCH, C), _F32),
            pltpu.VMEM((_CCH, C), _F32),
            pltpu.SemaphoreType.DMA,
            pltpu.SemaphoreType.DMA,
            pltpu.SemaphoreType.DMA,
            pltpu.SemaphoreType.DMA,
        ],
    )
    def _combine(yg_hbm, s1_hbm, s2_hbm, y1_hbm, y2_hbm, i0, i1, v0, v1,
                 g0, g1, w0, w1):
        sid = lax.axis_index("s")
        cid = lax.axis_index("c")
        wid = sid * NC + cid
        tbase = wid * _T_PER_W
        tasks = []
        for j in range(_NCCH):
            off = tbase + j * _CCH
            tasks.append((s1_hbm, y1_hbm, off))
            tasks.append((s2_hbm, y2_hbm, off))
        idxb = (i0, i1)
        rowb = (v0, v1)
        gsem = (g0, g1)
        wsem = (w0, w1)
        ghandles = [None, None]
        whandles = [None, None]
        nt = len(tasks)
        for t, (sh, yh, off) in enumerate(tasks):
            b = t % 2
            if whandles[b] is not None:
                whandles[b].wait()
            pltpu.sync_copy(sh.at[pl.ds(off, _CCH)], idxb[b])
            ghandles[b] = pltpu.async_copy(yg_hbm.at[idxb[b]], rowb[b],
                                           gsem[b])
            if t >= 1:
                pb = 1 - b
                psh, pyh, poff = tasks[t - 1]
                ghandles[pb].wait()
                whandles[pb] = pltpu.async_copy(
                    rowb[pb], pyh.at[pl.ds(poff, _CCH)], wsem[pb])
        lb = (nt - 1) % 2
        ghandles[lb].wait()
        lsh, lyh, loff = tasks[nt - 1]
        whandles[lb] = pltpu.async_copy(rowb[lb], lyh.at[pl.ds(loff, _CCH)],
                                        wsem[lb])
        whandles[0].wait()
        whandles[1].wait()

    return _combine


def _finish_kernel(y1_ref, y2_ref, w1n_ref, w2n_ref, out_ref):
    out_ref[...] = (w1n_ref[...] * y1_ref[...]
                    + w2n_ref[...] * y2_ref[...])


def _finish(y1, y2, w1n, w2n):
    return pl.pallas_call(
        _finish_kernel,
        grid=(NT,),
        in_specs=[
            pl.BlockSpec((TT, C), lambda i: (i, 0)),
            pl.BlockSpec((TT, C), lambda i: (i, 0)),
            pl.BlockSpec((TT, 1), lambda i: (i, 0)),
            pl.BlockSpec((TT, 1), lambda i: (i, 0)),
        ],
        out_specs=pl.BlockSpec((TT, C), lambda i: (i, 0)),
        out_shape=jax.ShapeDtypeStruct((T, C), _F32),
    )(y1, y2, w1n, w2n)


@jax.jit
def kernel(x_img, gate_w, w1, b1, w2, b2):
    x2d = x_img.reshape(T, D)
    e1, e2, w1n, w2n, bal, z, cnt = _gate(x2d, gate_w)
    esel = jnp.concatenate([e1, e2], axis=0)
    slots, be, act = _meta(cnt.reshape(E), esel.reshape(_MG, _MGW))
    slots_flat = slots.reshape(A)
    tokvals = jnp.tile(jnp.arange(T, dtype=_I32), 2)
    xg = _dispatch_kernel()(slots_flat, tokvals, x2d)
    yg = _mlp(be.reshape(NBLK), act.reshape(NBLK), xg, w1, b1, w2, b2)
    y1, y2 = _combine_kernel()(yg, slots_flat[:T], slots_flat[T:])
    out = _finish(y1, y2, w1n, w2n)
    return (out.reshape(B, N, C), bal[0, 0], z[0, 0])


# speedup vs baseline: 1.0906x; 1.0130x over previous
import functools

import jax
import jax.numpy as jnp
from jax import lax
from jax.experimental import pallas as pl
from jax.experimental.pallas import tpu as pltpu
from jax.experimental.pallas import tpu_sc as plsc

E = 8
K = 2
D = 1024
C = 1024
B = 2
N = 2048
T = B * N
A = K * T
BLK = 512
CAP = A + E * BLK
NBLK = CAP // BLK
TT = 512
NT = T // TT
TILES_PER_B = NT // B
AB = 512
NAB = A // AB

NC = 2
NS = 16
NW = NC * NS

_F32 = jnp.float32
_I32 = jnp.int32
_BF16 = jnp.bfloat16


def _erf(x):
    s = jnp.sign(x)
    a = jnp.abs(x)
    t = 1.0 / (1.0 + 0.3275911 * a)
    poly = t * (0.254829592
                + t * (-0.284496736
                       + t * (1.421413741
                              + t * (-1.453152027 + t * 1.061405429))))
    return s * (1.0 - poly * jnp.exp(-a * a))


def _gelu_exact(x):
    return 0.5 * x * (1.0 + _erf(x * 0.7071067811865476))


def _gate_kernel(x_ref, gw_ref, e1_ref, e2_ref, w1n_ref, w2n_ref, bal_ref,
                 z_ref, cnt_ref, proxy_acc, dens_acc, z_acc, cnt_acc):
    i = pl.program_id(0)
    x = x_ref[...]
    gw = gw_ref[...]
    logits = jax.lax.dot_general(x, gw, (((1,), (1,)), ((), ())),
                                 preferred_element_type=_F32)
    m = jnp.max(logits, axis=1, keepdims=True)
    ex = jnp.exp(logits - m)
    sumex = jnp.sum(ex, axis=1, keepdims=True)
    lse = m[:, 0] + jnp.log(sumex[:, 0])
    p = ex / sumex

    lane = jax.lax.broadcasted_iota(_I32, p.shape, 1)
    m1 = jnp.max(p, axis=1, keepdims=True)
    i1 = jnp.min(jnp.where(p == m1, lane, E), axis=1, keepdims=True)
    oh1 = lane == i1
    p2 = jnp.where(oh1, -jnp.inf, p)
    m2 = jnp.max(p2, axis=1, keepdims=True)
    i2 = jnp.min(jnp.where(p2 == m2, lane, E), axis=1, keepdims=True)
    oh2 = lane == i2
    denom = m1 + m2
    e1_ref[...] = i1
    e2_ref[...] = i2
    w1n_ref[...] = m1 / denom
    w2n_ref[...] = m2 / denom

    @pl.when(i == 0)
    def _():
        proxy_acc[...] = jnp.zeros_like(proxy_acc)
        dens_acc[...] = jnp.zeros_like(dens_acc)
        z_acc[...] = jnp.zeros_like(z_acc)
        cnt_acc[...] = jnp.zeros_like(cnt_acc)

    b = i // TILES_PER_B
    rows = jax.lax.broadcasted_iota(_I32, (B, E), 0)
    sel = (rows == b).astype(_F32)
    proxy_acc[...] += sel * jnp.sum(p, axis=0)[None, :]
    dens_acc[...] += sel * jnp.sum(oh1.astype(_F32), axis=0)[None, :]
    z_acc[...] += jnp.sum(lse * lse).reshape(1, 1)
    cnt_acc[...] += (jnp.sum(oh1.astype(_F32), axis=0)
                     + jnp.sum(oh2.astype(_F32), axis=0))[None, :]

    @pl.when(i == NT - 1)
    def _():
        proxy = proxy_acc[...] / jnp.float32(N)
        dens = dens_acc[...] / jnp.float32(N)
        bal_ref[...] = (jnp.sum(proxy * dens) / jnp.float32(B * E)
                        * jnp.float32(E * E)).reshape(1, 1)
        z_ref[...] = (z_acc[0, 0] / jnp.float32(T)).reshape(1, 1)
        cnt_ref[...] = cnt_acc[...].astype(_I32)


def _gate(x2d, gate_w):
    return pl.pallas_call(
        _gate_kernel,
        grid=(NT,),
        in_specs=[
            pl.BlockSpec((TT, D), lambda i: (i, 0)),
            pl.BlockSpec((E, D), lambda i: (0, 0)),
        ],
        out_specs=[
            pl.BlockSpec((TT, 1), lambda i: (i, 0)),
            pl.BlockSpec((TT, 1), lambda i: (i, 0)),
            pl.BlockSpec((TT, 1), lambda i: (i, 0)),
            pl.BlockSpec((TT, 1), lambda i: (i, 0)),
            pl.BlockSpec((1, 1), lambda i: (0, 0)),
            pl.BlockSpec((1, 1), lambda i: (0, 0)),
            pl.BlockSpec((1, E), lambda i: (0, 0)),
        ],
        out_shape=[
            jax.ShapeDtypeStruct((T, 1), _I32),
            jax.ShapeDtypeStruct((T, 1), _I32),
            jax.ShapeDtypeStruct((T, 1), _F32),
            jax.ShapeDtypeStruct((T, 1), _F32),
            jax.ShapeDtypeStruct((1, 1), _F32),
            jax.ShapeDtypeStruct((1, 1), _F32),
            jax.ShapeDtypeStruct((1, E), _I32),
        ],
        scratch_shapes=[
            pltpu.VMEM((B, E), _F32),
            pltpu.VMEM((B, E), _F32),
            pltpu.VMEM((1, 1), _F32),
            pltpu.VMEM((1, E), _F32),
        ],
    )(x2d, gate_w)


_MG = 16
_MGW = A // _MG


def _meta_kernel(cnt_ref, esel_ref, slots_ref, be_ref, act_ref):
    off = []
    run = 0
    for e in range(E):
        off.append(run)
        pad_e = ((cnt_ref[e] + (BLK - 1)) // BLK) * BLK
        run = run + pad_e
    starts = jax.lax.broadcasted_iota(_I32, (1, NBLK), 1) * BLK
    be = jnp.zeros((1, NBLK), _I32)
    maxe = 0
    for e in range(E):
        pad_e = ((cnt_ref[e] + (BLK - 1)) // BLK) * BLK
        end_e = off[e] + pad_e
        be = be + (starts >= end_e).astype(_I32)
        maxe = jnp.where(pad_e > 0, e, maxe)
    be_ref[...] = jnp.minimum(be, maxe)
    act_ref[...] = (starts < run).astype(_I32)

    r = jax.lax.broadcasted_iota(_I32, (_MGW, _MGW), 0)
    c = jax.lax.broadcasted_iota(_I32, (_MGW, _MGW), 1)
    triu = (r < c).astype(_F32)
    sub = jax.lax.broadcasted_iota(_I32, (E, _MGW), 0)
    runf = [jnp.float32(0.0)] * E
    for g in range(_MG):
        ev = esel_ref[g, :][None, :]
        onehot = (jnp.broadcast_to(ev, (E, _MGW)) == sub).astype(_F32)
        rank = jax.lax.dot_general(onehot, triu, (((1,), (0,)), ((), ())),
                                   preferred_element_type=_F32)
        rank_row = jnp.sum(rank * onehot, axis=0, keepdims=True)
        base = jnp.zeros((1, _MGW), _F32)
        for e in range(E):
            base = base + jnp.where(ev == e,
                                    jnp.float32(off[e]) + runf[e], 0.0)
            runf[e] = runf[e] + jnp.sum(onehot[e, :])
        slots_ref[g, :] = (base + rank_row).astype(_I32)[0, :]


def _meta(cnt, esel2d):
    grid_spec = pltpu.PrefetchScalarGridSpec(
        num_scalar_prefetch=1,
        grid=(1,),
        in_specs=[
            pl.BlockSpec((_MG, _MGW), lambda i, cnt: (0, 0)),
        ],
        out_specs=[
            pl.BlockSpec((_MG, _MGW), lambda i, cnt: (0, 0)),
            pl.BlockSpec((1, NBLK), lambda i, cnt: (0, 0)),
            pl.BlockSpec((1, NBLK), lambda i, cnt: (0, 0)),
        ],
    )
    return pl.pallas_call(
        _meta_kernel,
        grid_spec=grid_spec,
        out_shape=[
            jax.ShapeDtypeStruct((_MG, _MGW), _I32),
            jax.ShapeDtypeStruct((1, NBLK), _I32),
            jax.ShapeDtypeStruct((1, NBLK), _I32),
        ],
    )(cnt, esel2d)


_A_PER_W = A // NW
_DCH = 32
_NDCH = _A_PER_W // _DCH


@functools.cache
def _vmesh():
    return plsc.VectorSubcoreMesh(core_axis_name="c", subcore_axis_name="s")


@functools.cache
def _dispatch_kernel():
    @functools.partial(
        pl.kernel,
        mesh=_vmesh(),
        out_type=jax.ShapeDtypeStruct((CAP, D), _F32),
        scratch_types=[
            pltpu.VMEM((_DCH,), _I32),
            pltpu.VMEM((_DCH,), _I32),
            pltpu.VMEM((_DCH,), _I32),
            pltpu.VMEM((_DCH,), _I32),
            pltpu.VMEM((_DCH, D), _F32),
            pltpu.VMEM((_DCH, D), _F32),
            pltpu.SemaphoreType.DMA,
            pltpu.SemaphoreType.DMA,
            pltpu.SemaphoreType.DMA,
            pltpu.SemaphoreType.DMA,
        ],
    )
    def _dispatch(slots_hbm, x_hbm, xg_hbm, t0, t1, s0, s1,
                  rows0, rows1, g0, g1, w0, w1):
        sid = lax.axis_index("s")
        cid = lax.axis_index("c")
        wid = sid * NC + cid
        abase = wid * _A_PER_W
        xbase = lax.rem(abase, T)
        slotb = (s0, s1)
        rowb = (rows0, rows1)
        gsem = (g0, g1)
        wsem = (w0, w1)
        ghandles = [None, None]
        whandles = [None, None]
        for j in range(_NDCH):
            b = j % 2
            off = abase + j * _DCH
            if whandles[b] is not None:
                whandles[b].wait()
            pltpu.sync_copy(slots_hbm.at[pl.ds(off, _DCH)], slotb[b])
            ghandles[b] = pltpu.async_copy(
                x_hbm.at[pl.ds(xbase + j * _DCH, _DCH)], rowb[b], gsem[b])
            if j >= 1:
                pb = 1 - b
                ghandles[pb].wait()
                whandles[pb] = pltpu.async_copy(rowb[pb],
                                                xg_hbm.at[slotb[pb]],
                                                wsem[pb])
        lb = (_NDCH - 1) % 2
        ghandles[lb].wait()
        whandles[lb] = pltpu.async_copy(rowb[lb], xg_hbm.at[slotb[lb]],
                                        wsem[lb])
        whandles[0].wait()
        whandles[1].wait()

    return _dispatch


def _mlp_kernel(be_ref, act_ref, xg_ref, w1_ref, b1_ref, w2_ref, b2_ref,
                yg_ref):
    @pl.when(act_ref[pl.program_id(0)] == 1)
    def _():
        h = jax.lax.dot_general(xg_ref[...], w1_ref[0],
                                (((1,), (1,)), ((), ())),
                                preferred_element_type=_F32)
        h = h + b1_ref[0]
        h = _gelu_exact(h)
        o = jax.lax.dot_general(h, w2_ref[0], (((1,), (1,)), ((), ())),
                                preferred_element_type=_F32)
        yg_ref[...] = o + b2_ref[0]


def _mlp(block_expert, act, xg, w1, b1, w2, b2):
    grid_spec = pltpu.PrefetchScalarGridSpec(
        num_scalar_prefetch=2,
        grid=(NBLK,),
        in_specs=[
            pl.BlockSpec((BLK, D),
                         lambda i, be, act: (act[i] * i, 0)),
            pl.BlockSpec((1, C, D), lambda i, be, act: (be[i], 0, 0)),
            pl.BlockSpec((1, 1, C), lambda i, be, act: (be[i], 0, 0)),
            pl.BlockSpec((1, C, C), lambda i, be, act: (be[i], 0, 0)),
            pl.BlockSpec((1, 1, C), lambda i, be, act: (be[i], 0, 0)),
        ],
        out_specs=pl.BlockSpec(
            (BLK, C), lambda i, be, act: (jnp.where(act[i] == 1, i, NBLK), 0)),
    )
    return pl.pallas_call(
        _mlp_kernel,
        grid_spec=grid_spec,
        out_shape=jax.ShapeDtypeStruct((CAP + BLK, C), _F32),
    )(block_expert, act, xg, w1, b1.reshape(E, 1, C), w2, b2.reshape(E, 1, C))


_T_PER_W = T // NW
_CCH = 32
_NCCH = _T_PER_W // _CCH


@functools.cache
def _combine_kernel():
    @functools.partial(
        pl.kernel,
        mesh=_vmesh(),
        out_type=[
            jax.ShapeDtypeStruct((T, C), _F32),
            jax.ShapeDtypeStruct((T, C), _F32),
        ],
        scratch_types=[
            pltpu.VMEM((_CCH,), _I32),
            pltpu.VMEM((_CCH,), _I32),
            pltpu.VMEM((_CCH, C), _F32),
            pltpu.VMEM((_CCH, C), _F32),
            pltpu.SemaphoreType.DMA,
            pltpu.SemaphoreType.DMA,
            pltpu.SemaphoreType.DMA,
            pltpu.SemaphoreType.DMA,
        ],
    )
    def _combine(yg_hbm, s1_hbm, s2_hbm, y1_hbm, y2_hbm, i0, i1, v0, v1,
                 g0, g1, w0, w1):
        sid = lax.axis_index("s")
        cid = lax.axis_index("c")
        wid = sid * NC + cid
        tbase = wid * _T_PER_W
        tasks = []
        for j in range(_NCCH):
            off = tbase + j * _CCH
            tasks.append((s1_hbm, y1_hbm, off))
            tasks.append((s2_hbm, y2_hbm, off))
        idxb = (i0, i1)
        rowb = (v0, v1)
        gsem = (g0, g1)
        wsem = (w0, w1)
        ghandles = [None, None]
        whandles = [None, None]
        nt = len(tasks)
        for t, (sh, yh, off) in enumerate(tasks):
            b = t % 2
            if whandles[b] is not None:
                whandles[b].wait()
            pltpu.sync_copy(sh.at[pl.ds(off, _CCH)], idxb[b])
            ghandles[b] = pltpu.async_copy(yg_hbm.at[idxb[b]], rowb[b],
                                           gsem[b])
            if t >= 1:
                pb = 1 - b
                psh, pyh, poff = tasks[t - 1]
                ghandles[pb].wait()
                whandles[pb] = pltpu.async_copy(
                    rowb[pb], pyh.at[pl.ds(poff, _CCH)], wsem[pb])
        lb = (nt - 1) % 2
        ghandles[lb].wait()
        lsh, lyh, loff = tasks[nt - 1]
        whandles[lb] = pltpu.async_copy(rowb[lb], lyh.at[pl.ds(loff, _CCH)],
                                        wsem[lb])
        whandles[0].wait()
        whandles[1].wait()

    return _combine


def _finish_kernel(y1_ref, y2_ref, w1n_ref, w2n_ref, out_ref):
    out_ref[...] = (w1n_ref[...] * y1_ref[...]
                    + w2n_ref[...] * y2_ref[...])


def _finish(y1, y2, w1n, w2n):
    return pl.pallas_call(
        _finish_kernel,
        grid=(NT,),
        in_specs=[
            pl.BlockSpec((TT, C), lambda i: (i, 0)),
            pl.BlockSpec((TT, C), lambda i: (i, 0)),
            pl.BlockSpec((TT, 1), lambda i: (i, 0)),
            pl.BlockSpec((TT, 1), lambda i: (i, 0)),
        ],
        out_specs=pl.BlockSpec((TT, C), lambda i: (i, 0)),
        out_shape=jax.ShapeDtypeStruct((T, C), _F32),
    )(y1, y2, w1n, w2n)


@jax.jit
def kernel(x_img, gate_w, w1, b1, w2, b2):
    x2d = x_img.reshape(T, D)
    e1, e2, w1n, w2n, bal, z, cnt = _gate(x2d, gate_w)
    esel = jnp.concatenate([e1, e2], axis=0)
    slots, be, act = _meta(cnt.reshape(E), esel.reshape(_MG, _MGW))
    slots_flat = slots.reshape(A)
    xg = _dispatch_kernel()(slots_flat, x2d)
    yg = _mlp(be.reshape(NBLK), act.reshape(NBLK), xg, w1, b1, w2, b2)
    y1, y2 = _combine_kernel()(yg, slots_flat[:T], slots_flat[T:])
    out = _finish(y1, y2, w1n, w2n)
    return (out.reshape(B, N, C), bal[0, 0], z[0, 0])


# speedup vs baseline: 1.0912x; 1.0006x over previous
import functools

import jax
import jax.numpy as jnp
from jax import lax
from jax.experimental import pallas as pl
from jax.experimental.pallas import tpu as pltpu
from jax.experimental.pallas import tpu_sc as plsc

E = 8
K = 2
D = 1024
C = 1024
B = 2
N = 2048
T = B * N
A = K * T
BLK = 512
CAP = A + E * BLK
NBLK = CAP // BLK
TT = 512
NT = T // TT
TILES_PER_B = NT // B
AB = 512
NAB = A // AB

NC = 2
NS = 16
NW = NC * NS

_F32 = jnp.float32
_I32 = jnp.int32
_BF16 = jnp.bfloat16


def _erf(x):
    s = jnp.sign(x)
    a = jnp.abs(x)
    t = 1.0 / (1.0 + 0.3275911 * a)
    poly = t * (0.254829592
                + t * (-0.284496736
                       + t * (1.421413741
                              + t * (-1.453152027 + t * 1.061405429))))
    return s * (1.0 - poly * jnp.exp(-a * a))


def _gelu_exact(x):
    return 0.5 * x * (1.0 + _erf(x * 0.7071067811865476))


def _gate_kernel(x_ref, gw_ref, e1_ref, e2_ref, w1n_ref, w2n_ref, bal_ref,
                 z_ref, cnt_ref, proxy_acc, dens_acc, z_acc, cnt_acc):
    i = pl.program_id(0)
    x = x_ref[...]
    gw = gw_ref[...]
    logits = jax.lax.dot_general(x, gw, (((1,), (1,)), ((), ())),
                                 preferred_element_type=_F32)
    m = jnp.max(logits, axis=1, keepdims=True)
    ex = jnp.exp(logits - m)
    sumex = jnp.sum(ex, axis=1, keepdims=True)
    lse = m[:, 0] + jnp.log(sumex[:, 0])
    p = ex / sumex

    lane = jax.lax.broadcasted_iota(_I32, p.shape, 1)
    m1 = jnp.max(p, axis=1, keepdims=True)
    i1 = jnp.min(jnp.where(p == m1, lane, E), axis=1, keepdims=True)
    oh1 = lane == i1
    p2 = jnp.where(oh1, -jnp.inf, p)
    m2 = jnp.max(p2, axis=1, keepdims=True)
    i2 = jnp.min(jnp.where(p2 == m2, lane, E), axis=1, keepdims=True)
    oh2 = lane == i2
    denom = m1 + m2
    e1_ref[...] = i1
    e2_ref[...] = i2
    w1n_ref[...] = m1 / denom
    w2n_ref[...] = m2 / denom

    @pl.when(i == 0)
    def _():
        proxy_acc[...] = jnp.zeros_like(proxy_acc)
        dens_acc[...] = jnp.zeros_like(dens_acc)
        z_acc[...] = jnp.zeros_like(z_acc)
        cnt_acc[...] = jnp.zeros_like(cnt_acc)

    b = i // TILES_PER_B
    rows = jax.lax.broadcasted_iota(_I32, (B, E), 0)
    sel = (rows == b).astype(_F32)
    proxy_acc[...] += sel * jnp.sum(p, axis=0)[None, :]
    dens_acc[...] += sel * jnp.sum(oh1.astype(_F32), axis=0)[None, :]
    z_acc[...] += jnp.sum(lse * lse).reshape(1, 1)
    cnt_acc[...] += (jnp.sum(oh1.astype(_F32), axis=0)
                     + jnp.sum(oh2.astype(_F32), axis=0))[None, :]

    @pl.when(i == NT - 1)
    def _():
        proxy = proxy_acc[...] / jnp.float32(N)
        dens = dens_acc[...] / jnp.float32(N)
        bal_ref[...] = (jnp.sum(proxy * dens) / jnp.float32(B * E)
                        * jnp.float32(E * E)).reshape(1, 1)
        z_ref[...] = (z_acc[0, 0] / jnp.float32(T)).reshape(1, 1)
        cnt_ref[...] = cnt_acc[...].astype(_I32)


def _gate(x2d, gate_w):
    return pl.pallas_call(
        _gate_kernel,
        grid=(NT,),
        in_specs=[
            pl.BlockSpec((TT, D), lambda i: (i, 0)),
            pl.BlockSpec((E, D), lambda i: (0, 0)),
        ],
        out_specs=[
            pl.BlockSpec((TT, 1), lambda i: (i, 0)),
            pl.BlockSpec((TT, 1), lambda i: (i, 0)),
            pl.BlockSpec((TT, 1), lambda i: (i, 0)),
            pl.BlockSpec((TT, 1), lambda i: (i, 0)),
            pl.BlockSpec((1, 1), lambda i: (0, 0)),
            pl.BlockSpec((1, 1), lambda i: (0, 0)),
            pl.BlockSpec((1, E), lambda i: (0, 0)),
        ],
        out_shape=[
            jax.ShapeDtypeStruct((T, 1), _I32),
            jax.ShapeDtypeStruct((T, 1), _I32),
            jax.ShapeDtypeStruct((T, 1), _F32),
            jax.ShapeDtypeStruct((T, 1), _F32),
            jax.ShapeDtypeStruct((1, 1), _F32),
            jax.ShapeDtypeStruct((1, 1), _F32),
            jax.ShapeDtypeStruct((1, E), _I32),
        ],
        scratch_shapes=[
            pltpu.VMEM((B, E), _F32),
            pltpu.VMEM((B, E), _F32),
            pltpu.VMEM((1, 1), _F32),
            pltpu.VMEM((1, E), _F32),
        ],
    )(x2d, gate_w)


_MG = 16
_MGW = A // _MG


def _meta_kernel(cnt_ref, esel_ref, slots_ref, be_ref, act_ref):
    off = []
    run = 0
    for e in range(E):
        off.append(run)
        pad_e = ((cnt_ref[e] + (BLK - 1)) // BLK) * BLK
        run = run + pad_e
    starts = jax.lax.broadcasted_iota(_I32, (1, NBLK), 1) * BLK
    be = jnp.zeros((1, NBLK), _I32)
    maxe = 0
    for e in range(E):
        pad_e = ((cnt_ref[e] + (BLK - 1)) // BLK) * BLK
        end_e = off[e] + pad_e
        be = be + (starts >= end_e).astype(_I32)
        maxe = jnp.where(pad_e > 0, e, maxe)
    be_ref[...] = jnp.minimum(be, maxe)
    act_ref[...] = (starts < run).astype(_I32)

    r = jax.lax.broadcasted_iota(_I32, (_MGW, _MGW), 0)
    c = jax.lax.broadcasted_iota(_I32, (_MGW, _MGW), 1)
    triu = (r < c).astype(_F32)
    sub = jax.lax.broadcasted_iota(_I32, (E, _MGW), 0)
    runf = [jnp.float32(0.0)] * E
    for g in range(_MG):
        ev = esel_ref[g, :][None, :]
        onehot = (jnp.broadcast_to(ev, (E, _MGW)) == sub).astype(_F32)
        rank = jax.lax.dot_general(onehot, triu, (((1,), (0,)), ((), ())),
                                   preferred_element_type=_F32)
        rank_row = jnp.sum(rank * onehot, axis=0, keepdims=True)
        base = jnp.zeros((1, _MGW), _F32)
        for e in range(E):
            base = base + jnp.where(ev == e,
                                    jnp.float32(off[e]) + runf[e], 0.0)
            runf[e] = runf[e] + jnp.sum(onehot[e, :])
        slots_ref[g, :] = (base + rank_row).astype(_I32)[0, :]


def _meta(cnt, esel2d):
    grid_spec = pltpu.PrefetchScalarGridSpec(
        num_scalar_prefetch=1,
        grid=(1,),
        in_specs=[
            pl.BlockSpec((_MG, _MGW), lambda i, cnt: (0, 0)),
        ],
        out_specs=[
            pl.BlockSpec((_MG, _MGW), lambda i, cnt: (0, 0)),
            pl.BlockSpec((1, NBLK), lambda i, cnt: (0, 0)),
            pl.BlockSpec((1, NBLK), lambda i, cnt: (0, 0)),
        ],
    )
    return pl.pallas_call(
        _meta_kernel,
        grid_spec=grid_spec,
        out_shape=[
            jax.ShapeDtypeStruct((_MG, _MGW), _I32),
            jax.ShapeDtypeStruct((1, NBLK), _I32),
            jax.ShapeDtypeStruct((1, NBLK), _I32),
        ],
    )(cnt, esel2d)


_A_PER_W = A // NW
_DCH = 32
_NDCH = _A_PER_W // _DCH


@functools.cache
def _vmesh():
    return plsc.VectorSubcoreMesh(core_axis_name="c", subcore_axis_name="s")


@functools.cache
def _dispatch_kernel():
    @functools.partial(
        pl.kernel,
        mesh=_vmesh(),
        out_type=jax.ShapeDtypeStruct((CAP, D), _F32),
        scratch_types=[
            pltpu.VMEM((_DCH,), _I32),
            pltpu.VMEM((_DCH,), _I32),
            pltpu.VMEM((_DCH, D), _F32),
            pltpu.VMEM((_DCH, D), _F32),
            pltpu.SemaphoreType.DMA,
            pltpu.SemaphoreType.DMA,
            pltpu.SemaphoreType.DMA,
            pltpu.SemaphoreType.DMA,
        ],
    )
    def _dispatch(slots_hbm, x_hbm, xg_hbm, s0, s1,
                  rows0, rows1, g0, g1, w0, w1):
        sid = lax.axis_index("s")
        cid = lax.axis_index("c")
        wid = sid * NC + cid
        abase = wid * _A_PER_W
        xbase = lax.rem(abase, T)
        slotb = (s0, s1)
        rowb = (rows0, rows1)
        gsem = (g0, g1)
        wsem = (w0, w1)
        ghandles = [None, None]
        whandles = [None, None]
        for j in range(_NDCH):
            b = j % 2
            off = abase + j * _DCH
            if whandles[b] is not None:
                whandles[b].wait()
            pltpu.sync_copy(slots_hbm.at[pl.ds(off, _DCH)], slotb[b])
            ghandles[b] = pltpu.async_copy(
                x_hbm.at[pl.ds(xbase + j * _DCH, _DCH)], rowb[b], gsem[b])
            if j >= 1:
                pb = 1 - b
                ghandles[pb].wait()
                whandles[pb] = pltpu.async_copy(rowb[pb],
                                                xg_hbm.at[slotb[pb]],
                                                wsem[pb])
        lb = (_NDCH - 1) % 2
        ghandles[lb].wait()
        whandles[lb] = pltpu.async_copy(rowb[lb], xg_hbm.at[slotb[lb]],
                                        wsem[lb])
        whandles[0].wait()
        whandles[1].wait()

    return _dispatch


def _mlp_kernel(be_ref, act_ref, xg_ref, w1_ref, b1_ref, w2_ref, b2_ref,
                yg_ref):
    @pl.when(act_ref[pl.program_id(0)] == 1)
    def _():
        h = jax.lax.dot_general(xg_ref[...], w1_ref[0],
                                (((1,), (1,)), ((), ())),
                                preferred_element_type=_F32)
        h = h + b1_ref[0]
        h = _gelu_exact(h)
        o = jax.lax.dot_general(h, w2_ref[0], (((1,), (1,)), ((), ())),
                                preferred_element_type=_F32)
        yg_ref[...] = o + b2_ref[0]


def _mlp(block_expert, act, xg, w1, b1, w2, b2):
    grid_spec = pltpu.PrefetchScalarGridSpec(
        num_scalar_prefetch=2,
        grid=(NBLK,),
        in_specs=[
            pl.BlockSpec((BLK, D),
                         lambda i, be, act: (act[i] * i, 0)),
            pl.BlockSpec((1, C, D), lambda i, be, act: (be[i], 0, 0)),
            pl.BlockSpec((1, 1, C), lambda i, be, act: (be[i], 0, 0)),
            pl.BlockSpec((1, C, C), lambda i, be, act: (be[i], 0, 0)),
            pl.BlockSpec((1, 1, C), lambda i, be, act: (be[i], 0, 0)),
        ],
        out_specs=pl.BlockSpec(
            (BLK, C), lambda i, be, act: (jnp.where(act[i] == 1, i, NBLK), 0)),
    )
    return pl.pallas_call(
        _mlp_kernel,
        grid_spec=grid_spec,
        out_shape=jax.ShapeDtypeStruct((CAP + BLK, C), _F32),
    )(block_expert, act, xg, w1, b1.reshape(E, 1, C), w2, b2.reshape(E, 1, C))


_T_PER_W = T // NW
_CCH = 32
_NCCH = _T_PER_W // _CCH


@functools.cache
def _combine_kernel():
    @functools.partial(
        pl.kernel,
        mesh=_vmesh(),
        out_type=[
            jax.ShapeDtypeStruct((T, C), _F32),
            jax.ShapeDtypeStruct((T, C), _F32),
        ],
        scratch_types=[
            pltpu.VMEM((_CCH,), _I32),
            pltpu.VMEM((_CCH,), _I32),
            pltpu.VMEM((_CCH, C), _F32),
            pltpu.VMEM((_CCH, C), _F32),
            pltpu.SemaphoreType.DMA,
            pltpu.SemaphoreType.DMA,
            pltpu.SemaphoreType.DMA,
            pltpu.SemaphoreType.DMA,
        ],
    )
    def _combine(yg_hbm, s1_hbm, s2_hbm, y1_hbm, y2_hbm, i0, i1, v0, v1,
                 g0, g1, w0, w1):
        sid = lax.axis_index("s")
        cid = lax.axis_index("c")
        wid = sid * NC + cid
        tbase = wid * _T_PER_W
        tasks = []
        for j in range(_NCCH):
            off = tbase + j * _CCH
            tasks.append((s1_hbm, y1_hbm, off))
            tasks.append((s2_hbm, y2_hbm, off))
        idxb = (i0, i1)
        rowb = (v0, v1)
        gsem = (g0, g1)
        wsem = (w0, w1)
        ghandles = [None, None]
        whandles = [None, None]
        nt = len(tasks)
        for t, (sh, yh, off) in enumerate(tasks):
            b = t % 2
            if whandles[b] is not None:
                whandles[b].wait()
            pltpu.sync_copy(sh.at[pl.ds(off, _CCH)], idxb[b])
            ghandles[b] = pltpu.async_copy(yg_hbm.at[idxb[b]], rowb[b],
                                           gsem[b])
            if t >= 1:
                pb = 1 - b
                psh, pyh, poff = tasks[t - 1]
                ghandles[pb].wait()
                whandles[pb] = pltpu.async_copy(
                    rowb[pb], pyh.at[pl.ds(poff, _CCH)], wsem[pb])
        lb = (nt - 1) % 2
        ghandles[lb].wait()
        lsh, lyh, loff = tasks[nt - 1]
        whandles[lb] = pltpu.async_copy(rowb[lb], lyh.at[pl.ds(loff, _CCH)],
                                        wsem[lb])
        whandles[0].wait()
        whandles[1].wait()

    return _combine


def _finish_kernel(y1_ref, y2_ref, w1n_ref, w2n_ref, out_ref):
    out_ref[...] = (w1n_ref[...] * y1_ref[...]
                    + w2n_ref[...] * y2_ref[...])


def _finish(y1, y2, w1n, w2n):
    return pl.pallas_call(
        _finish_kernel,
        grid=(NT,),
        in_specs=[
            pl.BlockSpec((TT, C), lambda i: (i, 0)),
            pl.BlockSpec((TT, C), lambda i: (i, 0)),
            pl.BlockSpec((TT, 1), lambda i: (i, 0)),
            pl.BlockSpec((TT, 1), lambda i: (i, 0)),
        ],
        out_specs=pl.BlockSpec((TT, C), lambda i: (i, 0)),
        out_shape=jax.ShapeDtypeStruct((T, C), _F32),
    )(y1, y2, w1n, w2n)


@jax.jit
def kernel(x_img, gate_w, w1, b1, w2, b2):
    x2d = x_img.reshape(T, D)
    e1, e2, w1n, w2n, bal, z, cnt = _gate(x2d, gate_w)
    esel = jnp.concatenate([e1, e2], axis=0)
    slots, be, act = _meta(cnt.reshape(E), esel.reshape(_MG, _MGW))
    slots_flat = slots.reshape(A)
    xg = _dispatch_kernel()(slots_flat, x2d)
    yg = _mlp(be.reshape(NBLK), act.reshape(NBLK), xg, w1, b1, w2, b2)
    y1, y2 = _combine_kernel()(yg, slots_flat[:T], slots_flat[T:])
    out = _finish(y1, y2, w1n, w2n)
    return (out.reshape(B, N, C), bal[0, 0], z[0, 0])
